# Initial kernel scaffold; baseline (speedup 1.0000x reference)
import jax
import jax.numpy as jnp
from jax.experimental import pallas as pl


def kernel(features, mask, distance_matrix, tst_token, codebook, W1, b1, W2, b2):
    raise NotImplementedError("write your pallas kernel here")


# speedup vs baseline: 950.8893x; 950.8893x over previous
import jax
import jax.numpy as jnp
from jax.experimental import pallas as pl
from jax.experimental.pallas import tpu as pltpu

_B, _N, _DIN, _DH = 16, 1024, 128, 256


def _block_kernel(dm_ref, row0l_ref, row0c_ref, feat_ref, tst_ref,
                  cb_ref, w1_ref, b1_ref, w2_ref, b2_ref,
                  x_ref, bias_ref):
    h = jnp.dot(cb_ref[...], w1_ref[...], preferred_element_type=jnp.float32,
                precision=jax.lax.Precision.HIGHEST)
    h = jnp.maximum(h + b1_ref[...], 0.0)
    s = jnp.sum(h * w2_ref[...], axis=1, keepdims=True) + b2_ref[0, 0]
    s0, s1, s2, s3, s4, s5 = (s[0, 0], s[1, 0], s[2, 0], s[3, 0], s[4, 0], s[5, 0])

    d = dm_ref[0]
    f = jnp.where(d == 9999, s5,
        jnp.where(d == 0, s0,
        jnp.where(d == 1, s1,
        jnp.where(d == 2, s2,
        jnp.where(d == 3, s3,
        jnp.where(d == 4, s4, 0.0))))))
    bias_ref[0, 1:, 1:] = f

    def _border(r):
        return jnp.where(r == 9999, s5,
               jnp.where(r == 0, s1,
               jnp.where(r == 1, s2,
               jnp.where(r == 2, s3,
               jnp.where(r == 3, s4, 0.0)))))

    bias_ref[0, 0:1, 1:] = _border(row0l_ref[0])
    bias_ref[0, 1:, 0:1] = _border(row0c_ref[0])
    bias_ref[0, 0:1, 0:1] = jnp.broadcast_to(s0, (1, 1))

    x_ref[0, 0:1, :] = tst_ref[0]
    x_ref[0, 1:, :] = feat_ref[0]


def kernel(features, mask, distance_matrix, tst_token, codebook, W1, b1, W2, b2):
    b, n, d = features.shape
    dh = W1.shape[1]
    row0 = distance_matrix[:, 0:1, :]
    row0c = jnp.transpose(row0, (0, 2, 1))

    x_out, bias_out = pl.pallas_call(
        _block_kernel,
        grid=(b,),
        in_specs=[
            pl.BlockSpec((1, n, n), lambda i: (i, 0, 0)),
            pl.BlockSpec((1, 1, n), lambda i: (i, 0, 0)),
            pl.BlockSpec((1, n, 1), lambda i: (i, 0, 0)),
            pl.BlockSpec((1, n, d), lambda i: (i, 0, 0)),
            pl.BlockSpec((1, 1, d), lambda i: (0, 0, 0)),
            pl.BlockSpec((codebook.shape[0], d), lambda i: (0, 0)),
            pl.BlockSpec((d, dh), lambda i: (0, 0)),
            pl.BlockSpec((1, dh), lambda i: (0, 0)),
            pl.BlockSpec((1, dh), lambda i: (0, 0)),
            pl.BlockSpec((1, 1), lambda i: (0, 0)),
        ],
        out_specs=[
            pl.BlockSpec((1, n + 1, d), lambda i: (i, 0, 0)),
            pl.BlockSpec((1, n + 1, n + 1), lambda i: (i, 0, 0)),
        ],
        out_shape=[
            jax.ShapeDtypeStruct((b, n + 1, d), jnp.float32),
            jax.ShapeDtypeStruct((b, n + 1, n + 1), jnp.float32),
        ],
        compiler_params=pltpu.CompilerParams(
            dimension_semantics=("parallel",),
        ),
    )(distance_matrix, row0, row0c, features, tst_token,
      codebook, W1, b1.reshape(1, dh), W2.reshape(1, dh), b2.reshape(1, 1))

    input_mask = jnp.concatenate(
        [jnp.ones((b, 1), dtype=mask.dtype), mask], axis=1)
    return x_out, input_mask, bias_out


# speedup vs baseline: 1861.5588x; 1.9577x over previous
import jax
import jax.numpy as jnp
from jax.experimental import pallas as pl
from jax.experimental.pallas import tpu as pltpu

_RB = 104


def _block_kernel(dm_main, dm_prev8, r0t_main, r0t_prev8, feat_main, feat_prev8,
                  tst_ref, cb_ref, w1_ref, b1_ref, w2_ref, b2_ref,
                  x_ref, bias_ref):
    r = pl.program_id(0)
    first = r == 0
    dm_prev = jnp.where(first, dm_prev8[:, 0:1, :], dm_prev8[:, 7:8, :])
    r0t_prev = jnp.where(first, r0t_prev8[0:1, :], r0t_prev8[7:8, :])
    feat_prev = jnp.where(first, feat_prev8[:, 0:1, :], feat_prev8[:, 7:8, :])

    h = jnp.dot(cb_ref[...].astype(jnp.bfloat16), w1_ref[...].astype(jnp.bfloat16),
                preferred_element_type=jnp.float32)
    h = jnp.maximum(h + b1_ref[...], 0.0)
    h = h.astype(jnp.bfloat16).astype(jnp.float32)
    w2b = w2_ref[...].astype(jnp.bfloat16).astype(jnp.float32)
    s = jnp.sum(h * w2b, axis=1, keepdims=True) + b2_ref[0, 0]
    s0, s1, s2, s3, s4, s5 = (s[0, 0], s[1, 0], s[2, 0], s[3, 0], s[4, 0], s[5, 0])

    def _lut(d):
        return jnp.where(d == 9999, s5,
               jnp.where(d == 0, s0,
               jnp.where(d == 1, s1,
               jnp.where(d == 2, s2,
               jnp.where(d == 3, s3,
               jnp.where(d == 4, s4, 0.0))))))

    def _border(d):
        return jnp.where(d == 9999, s5,
               jnp.where(d == 0, s1,
               jnp.where(d == 1, s2,
               jnp.where(d == 2, s3,
               jnp.where(d == 3, s4, 0.0)))))

    shifted = jnp.concatenate(
        [dm_prev, dm_main[:, : _RB - 1, :]], axis=1)
    dmt = jnp.transpose(shifted, (1, 0, 2))
    bias_ref[:, :, 1:] = _lut(dmt)

    col_src = jnp.concatenate(
        [r0t_prev, r0t_main[: _RB - 1, :]], axis=0)
    bias_ref[:, :, 0] = _border(col_src)

    xs = jnp.concatenate(
        [feat_prev, feat_main[:, : _RB - 1, :]], axis=1)
    x_ref[...] = jnp.transpose(xs, (1, 0, 2))

    @pl.when(r == 0)
    def _():
        nb_, nd_ = feat_prev8.shape[0], feat_prev8.shape[2]
        bias_ref[0:1, :, 1:] = _border(jnp.transpose(dm_prev, (1, 0, 2)))
        bias_ref[0:1, :, 0] = jnp.broadcast_to(s0, (1, nb_))
        x_ref[0:1, :, :] = jnp.broadcast_to(tst_ref[...], (1, nb_, nd_))


def kernel(features, mask, distance_matrix, tst_token, codebook, W1, b1, W2, b2):
    b, n, d = features.shape
    dh = W1.shape[1]
    nb = -(-(n + 1) // _RB)
    r0t = jnp.transpose(distance_matrix[:, 0, :], (1, 0))

    x_t, bias_t = pl.pallas_call(
        _block_kernel,
        grid=(nb,),
        in_specs=[
            pl.BlockSpec((b, _RB, n), lambda r: (0, r, 0)),
            pl.BlockSpec((b, 8, n), lambda r: (0, jnp.maximum(_RB // 8 * r - 1, 0), 0)),
            pl.BlockSpec((_RB, b), lambda r: (r, 0)),
            pl.BlockSpec((8, b), lambda r: (jnp.maximum(_RB // 8 * r - 1, 0), 0)),
            pl.BlockSpec((b, _RB, d), lambda r: (0, r, 0)),
            pl.BlockSpec((b, 8, d), lambda r: (0, jnp.maximum(_RB // 8 * r - 1, 0), 0)),
            pl.BlockSpec((1, 1, d), lambda r: (0, 0, 0)),
            pl.BlockSpec((codebook.shape[0], d), lambda r: (0, 0)),
            pl.BlockSpec((d, dh), lambda r: (0, 0)),
            pl.BlockSpec((1, dh), lambda r: (0, 0)),
            pl.BlockSpec((1, dh), lambda r: (0, 0)),
            pl.BlockSpec((1, 1), lambda r: (0, 0)),
        ],
        out_specs=[
            pl.BlockSpec((_RB, b, d), lambda r: (r, 0, 0)),
            pl.BlockSpec((_RB, b, n + 1), lambda r: (r, 0, 0)),
        ],
        out_shape=[
            jax.ShapeDtypeStruct((n + 1, b, d), jnp.float32),
            jax.ShapeDtypeStruct((n + 1, b, n + 1), jnp.float32),
        ],
        compiler_params=pltpu.CompilerParams(
            dimension_semantics=("parallel",),
        ),
    )(distance_matrix, distance_matrix, r0t, r0t, features, features,
      tst_token, codebook, W1, b1.reshape(1, dh), W2.reshape(1, dh),
      b2.reshape(1, 1))

    input_x = jnp.transpose(x_t, (1, 0, 2))
    bias_mat = jnp.transpose(bias_t, (1, 0, 2))
    input_mask = jnp.concatenate(
        [jnp.ones((b, 1), dtype=mask.dtype), mask], axis=1)
    return input_x, input_mask, bias_mat
